# baseline (speedup 1.0000x reference)
import functools

import jax
import jax.numpy as jnp
from jax import lax
from jax.experimental import pallas as pl
from jax.experimental.pallas import tpu as pltpu
from jax.experimental.pallas import tpu_sc as plsc

N = 10000
E = 320000
IN = 128
HID = 16
HEADS = 8
OUT = 40

NPAD = 10240
PADROW = N
NCORES = 2
NSUB = 16
NTILES = NCORES * NSUB
B1 = 64
G1 = 168
B2 = 128
G2 = 84
C = B1 * G1
ET = NTILES * C
NPADROWS = 240

D1 = 144
D2 = 48
DD = 16

_SC_MESH = dict(core_axis_name="c", subcore_axis_name="s", num_cores=NCORES,
                num_subcores=NSUB)


def _tca_body(x_ref, w1_ref, msrc_ref, mdst_ref, ts1_ref, td1_ref):
    h = jnp.dot(x_ref[...], w1_ref[...], preferred_element_type=jnp.float32)
    a_src = jnp.dot(h, msrc_ref[...], preferred_element_type=jnp.float32)
    ts1_ref[...] = jnp.concatenate([h, a_src], axis=1)
    td1_ref[...] = jnp.dot(h, mdst_ref[...], preferred_element_type=jnp.float32)


def _tcb_body(acc_ref, rep_ref, b1_ref, w2_ref, p2_ref, q2_ref,
              ts2_ref, td2_ref):
    num = acc_ref[0, :, :IN] + acc_ref[1, :, :IN]
    den = acc_ref[0, :, IN:IN + HEADS] + acc_ref[1, :, IN:IN + HEADS]
    denx = jnp.dot(den, rep_ref[...], preferred_element_type=jnp.float32)
    out1 = num / (denx + 1e-16) + b1_ref[...]
    h1f = jnp.where(out1 > 0, out1, jnp.exp(out1) - 1.0)
    h2 = jnp.dot(h1f, w2_ref[...], preferred_element_type=jnp.float32)
    ts2_ref[...] = jnp.dot(h2, p2_ref[...], preferred_element_type=jnp.float32)
    td2_ref[...] = jnp.dot(h2, q2_ref[...], preferred_element_type=jnp.float32)


def _tcc_body(acc_ref, b2_ref, out_ref):
    num = acc_ref[0, :, :OUT] + acc_ref[1, :, :OUT]
    den = acc_ref[0, :, OUT:OUT + 1] + acc_ref[1, :, OUT:OUT + 1]
    logits = num / (den + 1e-16) + b2_ref[...]
    m = jnp.max(logits, axis=1, keepdims=True)
    s = logits - m
    lse = jnp.log(jnp.sum(jnp.exp(s), axis=1, keepdims=True))
    out_ref[...] = s - lse


def _sc_edge_body(*args, dcols, bb, gg, nb, compute_chunk):
    (ts_hbm, td_hbm, idx_hbm, out_hbm), rest = args[:4], list(args[4:])
    rows_b = tuple(rest[:nb]); del rest[:nb]
    adv_b = tuple(rest[:nb]); del rest[:nb]
    idx_v = rest.pop(0)
    acc_sh = rest.pop(0)
    gsem = tuple(rest[:nb]); del rest[:nb]
    ssem = tuple(rest[:nb]); del rest[:nb]
    isem = tuple(rest)

    cid = lax.axis_index("c")
    sid = lax.axis_index("s")
    tile = cid * NSUB + sid
    rpt = NPAD // NSUB
    lead = nb - 2

    def idx_issue(c, s6):
        pltpu.async_copy(idx_hbm.at[tile * gg + c], idx_v.at[s6], isem[s6])

    def idx_wait(c, s6):
        pltpu.make_async_copy(idx_hbm.at[tile * gg + c], idx_v.at[s6],
                              isem[s6]).wait()

    def g_issue(s6, b):
        pltpu.async_copy(ts_hbm.at[idx_v.at[s6, 0]], rows_b[b], gsem[b])
        pltpu.async_copy(td_hbm.at[idx_v.at[s6, 1]], adv_b[b], gsem[b])

    def g_wait(s6, b):
        pltpu.make_async_copy(ts_hbm.at[idx_v.at[s6, 0]], rows_b[b],
                              gsem[b]).wait()
        pltpu.make_async_copy(td_hbm.at[idx_v.at[s6, 1]], adv_b[b],
                              gsem[b]).wait()

    def s_issue(s6, b):
        pltpu.async_copy(rows_b[b], acc_sh.at[idx_v.at[s6, 1]], ssem[b],
                         add=True)

    def s_wait(s6, b):
        pltpu.make_async_copy(rows_b[b], acc_sh.at[idx_v.at[s6, 1]],
                              ssem[b]).wait()

    def step(g, u, first=False, no_next=False, no_idx=False):
        b = u % nb
        g_wait(u % 6, b)
        if not first:
            s_wait((u + 4) % 6, (u + nb - 2) % nb)
        if not no_next:
            idx_wait(g + lead, (u + lead) % 6)
            g_issue((u + lead) % 6, (u + lead) % nb)
        if not no_idx:
            idx_issue(g + lead + 2, (u + lead + 2) % 6)
        compute_chunk(rows_b[b], adv_b[b])
        s_issue(u % 6, b)

    for c in range(lead + 2):
        idx_issue(c, c)
    for c in range(lead):
        idx_wait(c, c)
        g_issue(c, c)

    zrows = rows_b[nb - 1]
    z16 = jnp.zeros((16,), jnp.float32)

    @pl.loop(0, bb)
    def _zero_rows(r):
        for c0 in range(dcols // 16):
            zrows[r, pl.ds(c0 * 16, 16)] = z16

    @pl.loop(0, rpt // bb)
    def _zero_acc(j):
        pltpu.sync_copy(zrows, acc_sh.at[pl.ds(sid * rpt + j * bb, bb)])

    if rpt % bb:
        pltpu.sync_copy(
            zrows.at[pl.ds(0, rpt % bb)],
            acc_sh.at[pl.ds(sid * rpt + (rpt // bb) * bb, rpt % bb)])

    plsc.subcore_barrier()

    P = 6 if nb == 3 else 12

    for u in range(P):
        step(u, u, first=(u < 2))

    @pl.loop(1, gg // P - 1)
    def _period(i):
        g0 = i * P
        for u in range(P):
            step(g0 + u, u % 12)

    for u in range(P):
        step(gg - P + u, u, no_next=(u >= P - lead),
             no_idx=(u >= P - 2 - lead))

    s_wait(4, (gg - 2) % nb)
    s_wait(5, (gg - 1) % nb)

    plsc.subcore_barrier()

    pltpu.sync_copy(acc_sh.at[pl.ds(sid * rpt, rpt)],
                    out_hbm.at[cid, pl.ds(sid * rpt, rpt)])


def _make_compute_l1(bb):
    def _compute(rows_v, adv_v):
        @pl.loop(0, bb, unroll=8)
        def _edge(e):
            as16 = rows_v[e, pl.ds(IN, 16)]
            ad16 = adv_v[e, pl.ds(0, 16)]
            al = as16 + ad16
            al = jnp.where(al > 0, al, 0.2 * al)
            w16 = jnp.exp(al)
            rows_v[e, pl.ds(IN, 16)] = w16
            for h in range(HEADS):
                wv = w16[h]
                rows_v[e, pl.ds(h * 16, 16)] = rows_v[e, pl.ds(h * 16, 16)] * wv
    return _compute


def _make_compute_l2(bb):
    def _compute(rows_v, adv_v):
        lane = lax.iota(jnp.int32, 16)
        is8 = lane == 8

        @pl.loop(0, bb, unroll=8)
        def _edge(e):
            r2 = rows_v[e, pl.ds(32, 16)]
            ad16 = adv_v[e, pl.ds(0, 16)]
            s = r2 + ad16
            al = jnp.where(s > 0, s, 0.2 * s)
            w16 = jnp.exp(al)
            wv = w16[8]
            rows_v[e, pl.ds(0, 16)] = rows_v[e, pl.ds(0, 16)] * wv
            rows_v[e, pl.ds(16, 16)] = rows_v[e, pl.ds(16, 16)] * wv
            rows_v[e, pl.ds(32, 16)] = jnp.where(is8, w16, r2 * wv)
    return _compute


def _make_sc_kernel(dcols, bb, gg, nb, compute_chunk, name):
    scratch = (
        [pltpu.VMEM((bb, dcols), jnp.float32)] * nb
        + [pltpu.VMEM((bb, DD), jnp.float32)] * nb
        + [pltpu.VMEM((6, 2, bb), jnp.int32)]
        + [pltpu.VMEM_SHARED((NPAD, dcols), jnp.float32)]
        + [pltpu.SemaphoreType.DMA] * (2 * nb + 6)
    )
    return pl.kernel(
        functools.partial(_sc_edge_body, dcols=dcols, bb=bb, gg=gg, nb=nb,
                          compute_chunk=compute_chunk),
        out_type=jax.ShapeDtypeStruct((NCORES, NPAD, dcols), jnp.float32),
        mesh=plsc.VectorSubcoreMesh(**_SC_MESH),
        scratch_types=scratch,
        compiler_params=pltpu.CompilerParams(use_tc_tiling_on_sc=False),
        name=name,
    )


def kernel(x, edge_index, W1, att_src1, att_dst1, b1, W2, att_src2, att_dst2,
           b2):
    f32 = jnp.float32
    import numpy as np
    tail = np.concatenate([
        np.arange(N, dtype=np.int32),
        PADROW + np.arange(ET - E - N, dtype=np.int32) % NPADROWS])
    src = jnp.concatenate([edge_index[0].astype(jnp.int32), jnp.asarray(tail)])
    dst = jnp.concatenate([edge_index[1].astype(jnp.int32), jnp.asarray(tail)])
    idx1 = jnp.stack([src.reshape(NTILES * G1, B1),
                      dst.reshape(NTILES * G1, B1)], axis=1)
    idx2 = jnp.stack([src.reshape(NTILES * G2, B2),
                      dst.reshape(NTILES * G2, B2)], axis=1)

    x_pad = jnp.zeros((NPAD, IN), f32).at[:N].set(x)

    eye8 = jnp.eye(HEADS, 16, dtype=f32)
    msrc = jnp.einsum("hd,hc->hdc", att_src1, eye8).reshape(IN, 16)
    mdst = jnp.einsum("hd,hc->hdc", att_dst1, eye8).reshape(IN, 16)
    rep = jnp.kron(jnp.eye(HEADS, dtype=f32), jnp.ones((1, HID), f32))
    p2 = jnp.concatenate(
        [jnp.eye(OUT, dtype=f32), att_src2.T, jnp.zeros((OUT, 7), f32)],
        axis=1)
    q2 = jnp.concatenate(
        [jnp.zeros((OUT, 8), f32), att_dst2.T, jnp.zeros((OUT, 7), f32)],
        axis=1)
    b1r = b1.reshape(1, IN)
    b2r = b2.reshape(1, OUT)

    RB = 2560
    grid = NPAD // RB

    ts1, td1 = pl.pallas_call(
        _tca_body,
        grid=(grid,),
        in_specs=[
            pl.BlockSpec((RB, IN), lambda i: (i, 0)),
            pl.BlockSpec((IN, IN), lambda i: (0, 0)),
            pl.BlockSpec((IN, 16), lambda i: (0, 0)),
            pl.BlockSpec((IN, 16), lambda i: (0, 0)),
        ],
        out_specs=[
            pl.BlockSpec((RB, D1), lambda i: (i, 0)),
            pl.BlockSpec((RB, DD), lambda i: (i, 0)),
        ],
        out_shape=[
            jax.ShapeDtypeStruct((NPAD, D1), f32),
            jax.ShapeDtypeStruct((NPAD, DD), f32),
        ],
    )(x_pad, W1, msrc, mdst)

    acc1 = _make_sc_kernel(D1, B1, G1, 3, _make_compute_l1(B1),
                           "sc_gat_l1")(ts1, td1, idx1)

    ts2, td2 = pl.pallas_call(
        _tcb_body,
        grid=(grid,),
        in_specs=[
            pl.BlockSpec((NCORES, RB, D1), lambda i: (0, i, 0)),
            pl.BlockSpec((HEADS, IN), lambda i: (0, 0)),
            pl.BlockSpec((1, IN), lambda i: (0, 0)),
            pl.BlockSpec((IN, OUT), lambda i: (0, 0)),
            pl.BlockSpec((OUT, D2), lambda i: (0, 0)),
            pl.BlockSpec((OUT, DD), lambda i: (0, 0)),
        ],
        out_specs=[
            pl.BlockSpec((RB, D2), lambda i: (i, 0)),
            pl.BlockSpec((RB, DD), lambda i: (i, 0)),
        ],
        out_shape=[
            jax.ShapeDtypeStruct((NPAD, D2), f32),
            jax.ShapeDtypeStruct((NPAD, DD), f32),
        ],
    )(acc1, rep, b1r, W2, p2, q2)

    acc2 = _make_sc_kernel(D2, B2, G2, 3, _make_compute_l2(B2),
                           "sc_gat_l2")(ts2, td2, idx2)

    out = pl.pallas_call(
        _tcc_body,
        grid=(grid,),
        in_specs=[
            pl.BlockSpec((NCORES, RB, D2), lambda i: (0, i, 0)),
            pl.BlockSpec((1, OUT), lambda i: (0, 0)),
        ],
        out_specs=pl.BlockSpec((RB, OUT), lambda i: (i, 0)),
        out_shape=jax.ShapeDtypeStruct((NPAD, OUT), f32),
    )(acc2, b2r)

    return out[:N]


# speedup vs baseline: 1.4646x; 1.4646x over previous
import functools

import jax
import jax.numpy as jnp
from jax import lax
from jax.experimental import pallas as pl
from jax.experimental.pallas import tpu as pltpu
from jax.experimental.pallas import tpu_sc as plsc

N = 10000
E = 320000
IN = 128
HID = 16
HEADS = 8
OUT = 40

NPAD = 10240
PADROW = N
NCORES = 2
NSUB = 16
NTILES = NCORES * NSUB
B1 = 64
G1 = 168
B2 = 128
G2 = 84
C = B1 * G1
ET = NTILES * C
NPADROWS = 240

D1 = 144
D2 = 48
DD = 16

_SC_MESH = dict(core_axis_name="c", subcore_axis_name="s", num_cores=NCORES,
                num_subcores=NSUB)


def _tca_body(x_ref, w1_ref, msrc_ref, mdst_ref, ts1_ref, td1_ref):
    h = jnp.dot(x_ref[...], w1_ref[...], preferred_element_type=jnp.float32)
    a_src = jnp.dot(h, msrc_ref[...], preferred_element_type=jnp.float32)
    ts1_ref[...] = jnp.concatenate([h, a_src], axis=1)
    td1_ref[...] = jnp.dot(h, mdst_ref[...], preferred_element_type=jnp.float32)


def _tcb_body(acc_ref, rep_ref, b1_ref, w2_ref, p2_ref, q2_ref,
              ts2_ref, td2_ref):
    num = acc_ref[0, :, :IN] + acc_ref[1, :, :IN]
    den = acc_ref[0, :, IN:IN + HEADS] + acc_ref[1, :, IN:IN + HEADS]
    denx = jnp.dot(den, rep_ref[...], preferred_element_type=jnp.float32)
    out1 = num / (denx + 1e-16) + b1_ref[...]
    h1f = jnp.where(out1 > 0, out1, jnp.exp(out1) - 1.0)
    h2 = jnp.dot(h1f, w2_ref[...], preferred_element_type=jnp.float32)
    ts2_ref[...] = jnp.dot(h2, p2_ref[...], preferred_element_type=jnp.float32)
    td2_ref[...] = jnp.dot(h2, q2_ref[...], preferred_element_type=jnp.float32)


def _tcc_body(acc_ref, b2_ref, out_ref):
    num = acc_ref[0, :, :OUT] + acc_ref[1, :, :OUT]
    den = acc_ref[0, :, OUT:OUT + 1] + acc_ref[1, :, OUT:OUT + 1]
    logits = num / (den + 1e-16) + b2_ref[...]
    m = jnp.max(logits, axis=1, keepdims=True)
    s = logits - m
    lse = jnp.log(jnp.sum(jnp.exp(s), axis=1, keepdims=True))
    out_ref[...] = s - lse


def _sc_edge_body(*args, dcols, bb, gg, nb, compute_chunk):
    (ts_hbm, td_hbm, idx_hbm, out_hbm), rest = args[:4], list(args[4:])
    rows_b = tuple(rest[:nb]); del rest[:nb]
    adv_b = tuple(rest[:nb]); del rest[:nb]
    idx_v = rest.pop(0)
    acc_sh = rest.pop(0)
    gsem = tuple(rest[:nb]); del rest[:nb]
    ssem = tuple(rest[:nb]); del rest[:nb]
    isem = tuple(rest)

    cid = lax.axis_index("c")
    sid = lax.axis_index("s")
    tile = cid * NSUB + sid
    rpt = NPAD // NSUB
    lead = nb - 2

    def idx_issue(c, s6):
        pltpu.async_copy(idx_hbm.at[tile * gg + c], idx_v.at[s6], isem[s6])

    def idx_wait(c, s6):
        pltpu.make_async_copy(idx_hbm.at[tile * gg + c], idx_v.at[s6],
                              isem[s6]).wait()

    def g_issue(s6, b):
        pltpu.async_copy(ts_hbm.at[idx_v.at[s6, 0]], rows_b[b], gsem[b])
        pltpu.async_copy(td_hbm.at[idx_v.at[s6, 1]], adv_b[b], gsem[b])

    def g_wait(s6, b):
        pltpu.make_async_copy(ts_hbm.at[idx_v.at[s6, 0]], rows_b[b],
                              gsem[b]).wait()
        pltpu.make_async_copy(td_hbm.at[idx_v.at[s6, 1]], adv_b[b],
                              gsem[b]).wait()

    def s_issue(s6, b):
        pltpu.async_copy(rows_b[b], acc_sh.at[idx_v.at[s6, 1]], ssem[b],
                         add=True)

    def s_wait(s6, b):
        pltpu.make_async_copy(rows_b[b], acc_sh.at[idx_v.at[s6, 1]],
                              ssem[b]).wait()

    def step(g, u, first=False, no_next=False, no_idx=False):
        b = u % nb
        g_wait(u % 6, b)
        if not first:
            s_wait((u + 4) % 6, (u + nb - 2) % nb)
        if not no_next:
            idx_wait(g + lead, (u + lead) % 6)
            g_issue((u + lead) % 6, (u + lead) % nb)
        if not no_idx:
            idx_issue(g + lead + 2, (u + lead + 2) % 6)
        compute_chunk(rows_b[b], adv_b[b])
        s_issue(u % 6, b)

    for c in range(lead + 2):
        idx_issue(c, c)
    for c in range(lead):
        idx_wait(c, c)
        g_issue(c, c)

    zrows = rows_b[nb - 1]
    z16 = jnp.zeros((16,), jnp.float32)

    @pl.loop(0, bb)
    def _zero_rows(r):
        for c0 in range(dcols // 16):
            zrows[r, pl.ds(c0 * 16, 16)] = z16

    @pl.loop(0, rpt // bb)
    def _zero_acc(j):
        pltpu.sync_copy(zrows, acc_sh.at[pl.ds(sid * rpt + j * bb, bb)])

    if rpt % bb:
        pltpu.sync_copy(
            zrows.at[pl.ds(0, rpt % bb)],
            acc_sh.at[pl.ds(sid * rpt + (rpt // bb) * bb, rpt % bb)])

    plsc.subcore_barrier()

    P = 6 if nb == 3 else 12

    for u in range(P):
        step(u, u, first=(u < 2))

    @pl.loop(1, gg // P - 1)
    def _period(i):
        g0 = i * P
        for u in range(P):
            step(g0 + u, u % 12)

    for u in range(P):
        step(gg - P + u, u, no_next=(u >= P - lead),
             no_idx=(u >= P - 2 - lead))

    s_wait(4, (gg - 2) % nb)
    s_wait(5, (gg - 1) % nb)

    plsc.subcore_barrier()

    pltpu.sync_copy(acc_sh.at[pl.ds(sid * rpt, rpt)],
                    out_hbm.at[cid, pl.ds(sid * rpt, rpt)])


def _make_compute_l1(bb):
    def _compute(rows_v, adv_v):
        @pl.loop(0, bb, unroll=4)
        def _edge(e):
            as16 = rows_v[e, pl.ds(IN, 16)]
            ad16 = adv_v[e, pl.ds(0, 16)]
            al = as16 + ad16
            al = jnp.where(al > 0, al, 0.2 * al)
            w16 = jnp.exp(al)
            rows_v[e, pl.ds(IN, 16)] = w16
            for h in range(HEADS):
                wv = w16[h]
                rows_v[e, pl.ds(h * 16, 16)] = rows_v[e, pl.ds(h * 16, 16)] * wv
    return _compute


def _make_compute_l2(bb):
    def _compute(rows_v, adv_v):
        lane = lax.iota(jnp.int32, 16)
        is8 = lane == 8

        @pl.loop(0, bb, unroll=4)
        def _edge(e):
            r2 = rows_v[e, pl.ds(32, 16)]
            ad16 = adv_v[e, pl.ds(0, 16)]
            s = r2 + ad16
            al = jnp.where(s > 0, s, 0.2 * s)
            w16 = jnp.exp(al)
            wv = w16[8]
            rows_v[e, pl.ds(0, 16)] = rows_v[e, pl.ds(0, 16)] * wv
            rows_v[e, pl.ds(16, 16)] = rows_v[e, pl.ds(16, 16)] * wv
            rows_v[e, pl.ds(32, 16)] = jnp.where(is8, w16, r2 * wv)
    return _compute


def _make_sc_kernel(dcols, bb, gg, nb, compute_chunk, name):
    scratch = (
        [pltpu.VMEM((bb, dcols), jnp.float32)] * nb
        + [pltpu.VMEM((bb, DD), jnp.float32)] * nb
        + [pltpu.VMEM((6, 2, bb), jnp.int32)]
        + [pltpu.VMEM_SHARED((NPAD, dcols), jnp.float32)]
        + [pltpu.SemaphoreType.DMA] * (2 * nb + 6)
    )
    return pl.kernel(
        functools.partial(_sc_edge_body, dcols=dcols, bb=bb, gg=gg, nb=nb,
                          compute_chunk=compute_chunk),
        out_type=jax.ShapeDtypeStruct((NCORES, NPAD, dcols), jnp.float32),
        mesh=plsc.VectorSubcoreMesh(**_SC_MESH),
        scratch_types=scratch,
        compiler_params=pltpu.CompilerParams(use_tc_tiling_on_sc=False),
        name=name,
    )


def kernel(x, edge_index, W1, att_src1, att_dst1, b1, W2, att_src2, att_dst2,
           b2):
    f32 = jnp.float32
    import numpy as np
    tail = np.concatenate([
        np.arange(N, dtype=np.int32),
        PADROW + np.arange(ET - E - N, dtype=np.int32) % NPADROWS])
    src = jnp.concatenate([edge_index[0].astype(jnp.int32), jnp.asarray(tail)])
    dst = jnp.concatenate([edge_index[1].astype(jnp.int32), jnp.asarray(tail)])
    idx1 = jnp.stack([src.reshape(NTILES * G1, B1),
                      dst.reshape(NTILES * G1, B1)], axis=1)
    idx2 = jnp.stack([src.reshape(NTILES * G2, B2),
                      dst.reshape(NTILES * G2, B2)], axis=1)

    x_pad = jnp.zeros((NPAD, IN), f32).at[:N].set(x)

    eye8 = jnp.eye(HEADS, 16, dtype=f32)
    msrc = jnp.einsum("hd,hc->hdc", att_src1, eye8).reshape(IN, 16)
    mdst = jnp.einsum("hd,hc->hdc", att_dst1, eye8).reshape(IN, 16)
    rep = jnp.kron(jnp.eye(HEADS, dtype=f32), jnp.ones((1, HID), f32))
    p2 = jnp.concatenate(
        [jnp.eye(OUT, dtype=f32), att_src2.T, jnp.zeros((OUT, 7), f32)],
        axis=1)
    q2 = jnp.concatenate(
        [jnp.zeros((OUT, 8), f32), att_dst2.T, jnp.zeros((OUT, 7), f32)],
        axis=1)
    b1r = b1.reshape(1, IN)
    b2r = b2.reshape(1, OUT)

    RB = 2560
    grid = NPAD // RB

    ts1, td1 = pl.pallas_call(
        _tca_body,
        grid=(grid,),
        in_specs=[
            pl.BlockSpec((RB, IN), lambda i: (i, 0)),
            pl.BlockSpec((IN, IN), lambda i: (0, 0)),
            pl.BlockSpec((IN, 16), lambda i: (0, 0)),
            pl.BlockSpec((IN, 16), lambda i: (0, 0)),
        ],
        out_specs=[
            pl.BlockSpec((RB, D1), lambda i: (i, 0)),
            pl.BlockSpec((RB, DD), lambda i: (i, 0)),
        ],
        out_shape=[
            jax.ShapeDtypeStruct((NPAD, D1), f32),
            jax.ShapeDtypeStruct((NPAD, DD), f32),
        ],
    )(x_pad, W1, msrc, mdst)

    acc1 = _make_sc_kernel(D1, B1, G1, 3, _make_compute_l1(B1),
                           "sc_gat_l1")(ts1, td1, idx1)

    ts2, td2 = pl.pallas_call(
        _tcb_body,
        grid=(grid,),
        in_specs=[
            pl.BlockSpec((NCORES, RB, D1), lambda i: (0, i, 0)),
            pl.BlockSpec((HEADS, IN), lambda i: (0, 0)),
            pl.BlockSpec((1, IN), lambda i: (0, 0)),
            pl.BlockSpec((IN, OUT), lambda i: (0, 0)),
            pl.BlockSpec((OUT, D2), lambda i: (0, 0)),
            pl.BlockSpec((OUT, DD), lambda i: (0, 0)),
        ],
        out_specs=[
            pl.BlockSpec((RB, D2), lambda i: (i, 0)),
            pl.BlockSpec((RB, DD), lambda i: (i, 0)),
        ],
        out_shape=[
            jax.ShapeDtypeStruct((NPAD, D2), f32),
            jax.ShapeDtypeStruct((NPAD, DD), f32),
        ],
    )(acc1, rep, b1r, W2, p2, q2)

    acc2 = _make_sc_kernel(D2, B2, G2, 3, _make_compute_l2(B2),
                           "sc_gat_l2")(ts2, td2, idx2)

    out = pl.pallas_call(
        _tcc_body,
        grid=(grid,),
        in_specs=[
            pl.BlockSpec((NCORES, RB, D2), lambda i: (0, i, 0)),
            pl.BlockSpec((1, OUT), lambda i: (0, 0)),
        ],
        out_specs=pl.BlockSpec((RB, OUT), lambda i: (i, 0)),
        out_shape=jax.ShapeDtypeStruct((NPAD, OUT), f32),
    )(acc2, b2r)

    return out[:N]


# speedup vs baseline: 1.4811x; 1.0113x over previous
import functools

import jax
import jax.numpy as jnp
from jax import lax
from jax.experimental import pallas as pl
from jax.experimental.pallas import tpu as pltpu
from jax.experimental.pallas import tpu_sc as plsc

N = 10000
E = 320000
IN = 128
HID = 16
HEADS = 8
OUT = 40

NPAD = 10240
PADROW = N
NCORES = 2
NSUB = 16
NTILES = NCORES * NSUB
B1 = 64
G1 = 168
B2 = 128
G2 = 84
C = B1 * G1
ET = NTILES * C
NPADROWS = 240

D1 = 144
D2 = 48
DD = 16

_SC_MESH = dict(core_axis_name="c", subcore_axis_name="s", num_cores=NCORES,
                num_subcores=NSUB)


def _tca_body(x_ref, w1_ref, msrc_ref, mdst_ref, ts1_ref, td1_ref):
    h = jnp.dot(x_ref[...], w1_ref[...], preferred_element_type=jnp.float32)
    a_src = jnp.dot(h, msrc_ref[...], preferred_element_type=jnp.float32)
    ts1_ref[...] = jnp.concatenate([h, a_src], axis=1)
    td1_ref[...] = jnp.dot(h, mdst_ref[...], preferred_element_type=jnp.float32)


def _tcb_body(acc_ref, rep_ref, b1_ref, w2_ref, p2_ref, q2_ref,
              ts2_ref, td2_ref):
    num = acc_ref[0, :, :IN] + acc_ref[1, :, :IN]
    den = acc_ref[0, :, IN:IN + HEADS] + acc_ref[1, :, IN:IN + HEADS]
    denx = jnp.dot(den, rep_ref[...], preferred_element_type=jnp.float32)
    out1 = num / (denx + 1e-16) + b1_ref[...]
    h1f = jnp.where(out1 > 0, out1, jnp.exp(out1) - 1.0)
    h2 = jnp.dot(h1f, w2_ref[...], preferred_element_type=jnp.float32)
    ts2_ref[...] = jnp.dot(h2, p2_ref[...], preferred_element_type=jnp.float32)
    td2_ref[...] = jnp.dot(h2, q2_ref[...], preferred_element_type=jnp.float32)


def _tcc_body(acc_ref, b2_ref, out_ref):
    num = acc_ref[0, :, :OUT] + acc_ref[1, :, :OUT]
    den = acc_ref[0, :, OUT:OUT + 1] + acc_ref[1, :, OUT:OUT + 1]
    logits = num / (den + 1e-16) + b2_ref[...]
    m = jnp.max(logits, axis=1, keepdims=True)
    s = logits - m
    lse = jnp.log(jnp.sum(jnp.exp(s), axis=1, keepdims=True))
    out_ref[...] = s - lse


def _sc_edge_body(*args, dcols, bb, gg, nb, compute_chunk):
    (ts_hbm, td_hbm, idx_hbm, out_hbm), rest = args[:4], list(args[4:])
    rows_b = tuple(rest[:nb]); del rest[:nb]
    adv_b = tuple(rest[:nb]); del rest[:nb]
    idx_v = rest.pop(0)
    acc_sh = rest.pop(0)
    gsem = tuple(rest[:nb]); del rest[:nb]
    ssem = tuple(rest[:nb]); del rest[:nb]
    isem = tuple(rest)

    cid = lax.axis_index("c")
    sid = lax.axis_index("s")
    tile = cid * NSUB + sid
    rpt = NPAD // NSUB
    lead = nb - 2

    def idx_issue(c, s6):
        pltpu.async_copy(idx_hbm.at[tile * gg + c], idx_v.at[s6], isem[s6])

    def idx_wait(c, s6):
        pltpu.make_async_copy(idx_hbm.at[tile * gg + c], idx_v.at[s6],
                              isem[s6]).wait()

    def g_issue(s6, b):
        pltpu.async_copy(ts_hbm.at[idx_v.at[s6, 0]], rows_b[b], gsem[b])
        pltpu.async_copy(td_hbm.at[idx_v.at[s6, 1]], adv_b[b], gsem[b])

    def g_wait(s6, b):
        pltpu.make_async_copy(ts_hbm.at[idx_v.at[s6, 0]], rows_b[b],
                              gsem[b]).wait()
        pltpu.make_async_copy(td_hbm.at[idx_v.at[s6, 1]], adv_b[b],
                              gsem[b]).wait()

    def s_issue(s6, b):
        pltpu.async_copy(rows_b[b], acc_sh.at[idx_v.at[s6, 1]], ssem[b],
                         add=True)

    def s_wait(s6, b):
        pltpu.make_async_copy(rows_b[b], acc_sh.at[idx_v.at[s6, 1]],
                              ssem[b]).wait()

    def step(g, u, first=False, no_next=False, no_idx=False):
        b = u % nb
        g_wait(u % 6, b)
        if not first:
            s_wait((u + 4) % 6, (u + nb - 2) % nb)
        if not no_next:
            idx_wait(g + lead, (u + lead) % 6)
            g_issue((u + lead) % 6, (u + lead) % nb)
        if not no_idx:
            idx_issue(g + lead + 2, (u + lead + 2) % 6)
        compute_chunk(rows_b[b], adv_b[b])
        s_issue(u % 6, b)

    for c in range(lead + 2):
        idx_issue(c, c)
    for c in range(lead):
        idx_wait(c, c)
        g_issue(c, c)

    zrows = rows_b[nb - 1]
    z16 = jnp.zeros((16,), jnp.float32)

    @pl.loop(0, bb)
    def _zero_rows(r):
        for c0 in range(dcols // 16):
            zrows[r, pl.ds(c0 * 16, 16)] = z16

    @pl.loop(0, rpt // bb)
    def _zero_acc(j):
        pltpu.sync_copy(zrows, acc_sh.at[pl.ds(sid * rpt + j * bb, bb)])

    if rpt % bb:
        pltpu.sync_copy(
            zrows.at[pl.ds(0, rpt % bb)],
            acc_sh.at[pl.ds(sid * rpt + (rpt // bb) * bb, rpt % bb)])

    plsc.subcore_barrier()

    P = 6 if nb == 3 else 12

    for u in range(P):
        step(u, u, first=(u < 2))

    @pl.loop(1, gg // P - 1)
    def _period(i):
        g0 = i * P
        for u in range(P):
            step(g0 + u, u % 12)

    for u in range(P):
        step(gg - P + u, u, no_next=(u >= P - lead),
             no_idx=(u >= P - 2 - lead))

    s_wait(4, (gg - 2) % nb)
    s_wait(5, (gg - 1) % nb)

    plsc.subcore_barrier()

    pltpu.sync_copy(acc_sh.at[pl.ds(sid * rpt, rpt)],
                    out_hbm.at[cid, pl.ds(sid * rpt, rpt)])


def _make_compute_l1(bb):
    def _compute(rows_v, adv_v):
        @pl.loop(0, bb, unroll=2)
        def _edge(e):
            as16 = rows_v[e, pl.ds(IN, 16)]
            ad16 = adv_v[e, pl.ds(0, 16)]
            al = as16 + ad16
            al = jnp.where(al > 0, al, 0.2 * al)
            w16 = jnp.exp(al)
            rows_v[e, pl.ds(IN, 16)] = w16
            for h in range(HEADS):
                wv = w16[h]
                rows_v[e, pl.ds(h * 16, 16)] = rows_v[e, pl.ds(h * 16, 16)] * wv
    return _compute


def _make_compute_l2(bb):
    def _compute(rows_v, adv_v):
        lane = lax.iota(jnp.int32, 16)
        is8 = lane == 8

        @pl.loop(0, bb, unroll=2)
        def _edge(e):
            r2 = rows_v[e, pl.ds(32, 16)]
            ad16 = adv_v[e, pl.ds(0, 16)]
            s = r2 + ad16
            al = jnp.where(s > 0, s, 0.2 * s)
            w16 = jnp.exp(al)
            wv = w16[8]
            rows_v[e, pl.ds(0, 16)] = rows_v[e, pl.ds(0, 16)] * wv
            rows_v[e, pl.ds(16, 16)] = rows_v[e, pl.ds(16, 16)] * wv
            rows_v[e, pl.ds(32, 16)] = jnp.where(is8, w16, r2 * wv)
    return _compute


def _make_sc_kernel(dcols, bb, gg, nb, compute_chunk, name):
    scratch = (
        [pltpu.VMEM((bb, dcols), jnp.float32)] * nb
        + [pltpu.VMEM((bb, DD), jnp.float32)] * nb
        + [pltpu.VMEM((6, 2, bb), jnp.int32)]
        + [pltpu.VMEM_SHARED((NPAD, dcols), jnp.float32)]
        + [pltpu.SemaphoreType.DMA] * (2 * nb + 6)
    )
    return pl.kernel(
        functools.partial(_sc_edge_body, dcols=dcols, bb=bb, gg=gg, nb=nb,
                          compute_chunk=compute_chunk),
        out_type=jax.ShapeDtypeStruct((NCORES, NPAD, dcols), jnp.float32),
        mesh=plsc.VectorSubcoreMesh(**_SC_MESH),
        scratch_types=scratch,
        compiler_params=pltpu.CompilerParams(use_tc_tiling_on_sc=False),
        name=name,
    )


def kernel(x, edge_index, W1, att_src1, att_dst1, b1, W2, att_src2, att_dst2,
           b2):
    f32 = jnp.float32
    import numpy as np
    tail = np.concatenate([
        np.arange(N, dtype=np.int32),
        PADROW + np.arange(ET - E - N, dtype=np.int32) % NPADROWS])
    src = jnp.concatenate([edge_index[0].astype(jnp.int32), jnp.asarray(tail)])
    dst = jnp.concatenate([edge_index[1].astype(jnp.int32), jnp.asarray(tail)])
    idx1 = jnp.stack([src.reshape(NTILES * G1, B1),
                      dst.reshape(NTILES * G1, B1)], axis=1)
    idx2 = jnp.stack([src.reshape(NTILES * G2, B2),
                      dst.reshape(NTILES * G2, B2)], axis=1)

    x_pad = jnp.zeros((NPAD, IN), f32).at[:N].set(x)

    eye8 = jnp.eye(HEADS, 16, dtype=f32)
    msrc = jnp.einsum("hd,hc->hdc", att_src1, eye8).reshape(IN, 16)
    mdst = jnp.einsum("hd,hc->hdc", att_dst1, eye8).reshape(IN, 16)
    rep = jnp.kron(jnp.eye(HEADS, dtype=f32), jnp.ones((1, HID), f32))
    p2 = jnp.concatenate(
        [jnp.eye(OUT, dtype=f32), att_src2.T, jnp.zeros((OUT, 7), f32)],
        axis=1)
    q2 = jnp.concatenate(
        [jnp.zeros((OUT, 8), f32), att_dst2.T, jnp.zeros((OUT, 7), f32)],
        axis=1)
    b1r = b1.reshape(1, IN)
    b2r = b2.reshape(1, OUT)

    RB = 2560
    grid = NPAD // RB

    ts1, td1 = pl.pallas_call(
        _tca_body,
        grid=(grid,),
        in_specs=[
            pl.BlockSpec((RB, IN), lambda i: (i, 0)),
            pl.BlockSpec((IN, IN), lambda i: (0, 0)),
            pl.BlockSpec((IN, 16), lambda i: (0, 0)),
            pl.BlockSpec((IN, 16), lambda i: (0, 0)),
        ],
        out_specs=[
            pl.BlockSpec((RB, D1), lambda i: (i, 0)),
            pl.BlockSpec((RB, DD), lambda i: (i, 0)),
        ],
        out_shape=[
            jax.ShapeDtypeStruct((NPAD, D1), f32),
            jax.ShapeDtypeStruct((NPAD, DD), f32),
        ],
    )(x_pad, W1, msrc, mdst)

    acc1 = _make_sc_kernel(D1, B1, G1, 3, _make_compute_l1(B1),
                           "sc_gat_l1")(ts1, td1, idx1)

    ts2, td2 = pl.pallas_call(
        _tcb_body,
        grid=(grid,),
        in_specs=[
            pl.BlockSpec((NCORES, RB, D1), lambda i: (0, i, 0)),
            pl.BlockSpec((HEADS, IN), lambda i: (0, 0)),
            pl.BlockSpec((1, IN), lambda i: (0, 0)),
            pl.BlockSpec((IN, OUT), lambda i: (0, 0)),
            pl.BlockSpec((OUT, D2), lambda i: (0, 0)),
            pl.BlockSpec((OUT, DD), lambda i: (0, 0)),
        ],
        out_specs=[
            pl.BlockSpec((RB, D2), lambda i: (i, 0)),
            pl.BlockSpec((RB, DD), lambda i: (i, 0)),
        ],
        out_shape=[
            jax.ShapeDtypeStruct((NPAD, D2), f32),
            jax.ShapeDtypeStruct((NPAD, DD), f32),
        ],
    )(acc1, rep, b1r, W2, p2, q2)

    acc2 = _make_sc_kernel(D2, B2, G2, 3, _make_compute_l2(B2),
                           "sc_gat_l2")(ts2, td2, idx2)

    out = pl.pallas_call(
        _tcc_body,
        grid=(grid,),
        in_specs=[
            pl.BlockSpec((NCORES, RB, D2), lambda i: (0, i, 0)),
            pl.BlockSpec((1, OUT), lambda i: (0, 0)),
        ],
        out_specs=pl.BlockSpec((RB, OUT), lambda i: (i, 0)),
        out_shape=jax.ShapeDtypeStruct((NPAD, OUT), f32),
    )(acc2, b2r)

    return out[:N]
